# baseline (speedup 1.0000x reference)
import jax
import jax.numpy as jnp
import numpy as np
from jax.experimental import pallas as pl

N = 10000
D = 86
NB = 10
MAXR = 2.5


def _sh(u):
    x = u[:, 0]; y = u[:, 1]; z = u[:, 2]
    x2 = x * x; y2 = y * y; z2 = z * z
    return jnp.stack([
        jnp.ones_like(x),
        x, y, z,
        x * y, y * z, 0.5 * (3.0 * z2 - 1.0), z * x, 0.5 * (x2 - y2),
        y * (3.0 * x2 - y2), x * y * z, y * (5.0 * z2 - 1.0),
        z * (5.0 * z2 - 3.0), x * (5.0 * z2 - 1.0), z * (x2 - y2),
        x * (x2 - 3.0 * y2),
    ], axis=-1)


def _rbf(d):
    centers = jnp.linspace(0.0, MAXR, NB)
    w = MAXR / NB
    g = jnp.exp(-(((d[:, None] - centers[None, :]) / w) ** 2))
    env = jnp.exp(-(d ** 2) / (2.0 * MAXR * MAXR))
    return g * env[:, None]


def _layer(x, src, dst, rbf, sh, Wq, Wk, Wv, Wr, Wsh, Wo):
    q = x @ Wq
    k = x @ Wk
    v = x @ Wv
    rmod = rbf @ Wr
    smod = sh @ Wsh
    ke = k[src] * rmod
    ve = v[src] * rmod + smod
    logits = jnp.sum(q[dst] * ke, axis=-1) / np.sqrt(D).astype(np.float32)
    m = jax.ops.segment_max(logits, dst, num_segments=N)
    m = jnp.where(jnp.isfinite(m), m, 0.0)
    ex = jnp.exp(logits - m[dst])
    den = jax.ops.segment_sum(ex, dst, num_segments=N) + 1e-9
    alpha = ex / den[dst]
    agg = jax.ops.segment_sum(alpha[:, None] * ve, dst, num_segments=N)
    x = x + agg @ Wo
    mu = jnp.mean(x, axis=-1, keepdims=True)
    sig = jnp.std(x, axis=-1, keepdims=True) + 1e-5
    return (x - mu) / sig


def _mm_kernel(x_ref, w_ref, o_ref):
    o_ref[...] = jnp.dot(x_ref[...], w_ref[...],
                         preferred_element_type=jnp.float32)


def _pallas_mm(x, w):
    blk = 1000
    return pl.pallas_call(
        _mm_kernel,
        grid=(N // blk,),
        in_specs=[pl.BlockSpec((blk, D), lambda i: (i, 0)),
                  pl.BlockSpec((D, D), lambda i: (0, 0))],
        out_specs=pl.BlockSpec((blk, D), lambda i: (i, 0)),
        out_shape=jax.ShapeDtypeStruct((N, D), jnp.float32),
    )(x, w)


def kernel(pos, edge_index, Wemb, Wq, Wk, Wv, Wr, Wsh, Wo, Wout):
    src = edge_index[0]
    dst = edge_index[1]
    rel = pos[dst] - pos[src]
    dist = jnp.linalg.norm(rel, axis=-1) + 1e-9
    u = rel / dist[:, None]
    rbf = _rbf(dist)
    sh = _sh(u)
    x = jnp.tanh(pos @ Wemb)
    for i in range(3):
        x = _layer(x, src, dst, rbf, sh, Wq[i], Wk[i], Wv[i], Wr[i], Wsh[i], Wo[i])
    return _pallas_mm(x, Wout)


# speedup vs baseline: 3.2829x; 3.2829x over previous
import functools

import jax
import jax.numpy as jnp
import numpy as np
from jax import lax
from jax.experimental import pallas as pl
from jax.experimental.pallas import tpu as pltpu
from jax.experimental.pallas import tpu_sc as plsc

N = 10000
E = 320000
D = 86
DP = 96
KVW = 2 * DP
RW = 104
NB = 10
MAXR = 2.5
L = 3

NC = 2
NS = 16
NW = NC * NS
EPW = E // NW
CH = 80
NJ = EPW // CH
STRIPE = N // NS

BE = 3200
GE = E // BE
BN = 2000
GN = N // BN

_SQRT_D = np.sqrt(D).astype(np.float32)


def _mesh():
    return plsc.VectorSubcoreMesh(core_axis_name="c", subcore_axis_name="s")


_SC_PARAMS = pltpu.CompilerParams(use_tc_tiling_on_sc=False)


def _sc_gather2(table_a, idx_a, table_b, idx_b):
    da = table_a.shape[1]
    db = table_b.shape[1]

    @functools.partial(
        pl.kernel,
        mesh=_mesh(),
        compiler_params=_SC_PARAMS,
        out_type=[
            jax.ShapeDtypeStruct((E, da), jnp.float32),
            jax.ShapeDtypeStruct((E, db), jnp.float32),
        ],
        scratch_types=[
            pltpu.VMEM((CH,), jnp.int32),
            pltpu.VMEM((CH, da), jnp.float32),
            pltpu.VMEM((CH,), jnp.int32),
            pltpu.VMEM((CH, db), jnp.float32),
            pltpu.SemaphoreType.DMA,
            pltpu.SemaphoreType.DMA,
        ],
    )
    def gk(ta, ia, tb, ib, oa, ob, ibufa, rowsa, ibufb, rowsb, sema, semb):
        cc = lax.axis_index("c")
        ss = lax.axis_index("s")
        base = (ss * NC + cc) * EPW

        def step(j, carry):
            off = base + j * CH
            pltpu.sync_copy(ia.at[pl.ds(off, CH)], ibufa)
            pltpu.sync_copy(ib.at[pl.ds(off, CH)], ibufb)
            ca = pltpu.async_copy(ta.at[ibufa], rowsa, sema)
            cb = pltpu.async_copy(tb.at[ibufb], rowsb, semb)
            ca.wait()
            cb.wait()
            pltpu.sync_copy(rowsa, oa.at[pl.ds(off, CH)])
            pltpu.sync_copy(rowsb, ob.at[pl.ds(off, CH)])
            return carry

        lax.fori_loop(0, NJ, step, 0)

    return gk(table_a, idx_a, table_b, idx_b)


def _sc_gather_qkv(qt, kt, vt, dst, src):
    @functools.partial(
        pl.kernel,
        mesh=_mesh(),
        compiler_params=_SC_PARAMS,
        out_type=[
            jax.ShapeDtypeStruct((E, DP), jnp.float32),
            jax.ShapeDtypeStruct((E, DP), jnp.float32),
            jax.ShapeDtypeStruct((E, DP), jnp.float32),
        ],
        scratch_types=[
            pltpu.VMEM((CH,), jnp.int32),
            pltpu.VMEM((CH,), jnp.int32),
            pltpu.VMEM((CH, DP), jnp.float32),
            pltpu.VMEM((CH, DP), jnp.float32),
            pltpu.VMEM((CH, DP), jnp.float32),
            pltpu.SemaphoreType.DMA,
            pltpu.SemaphoreType.DMA,
            pltpu.SemaphoreType.DMA,
        ],
    )
    def gk(tq, tk, tv, ds_, sr_, oq, ok, ov, ibd, ibs, rq, rk, rv, s0, s1, s2):
        cc = lax.axis_index("c")
        ss = lax.axis_index("s")
        base = (ss * NC + cc) * EPW

        def step(j, carry):
            off = base + j * CH
            pltpu.sync_copy(ds_.at[pl.ds(off, CH)], ibd)
            pltpu.sync_copy(sr_.at[pl.ds(off, CH)], ibs)
            c0 = pltpu.async_copy(tq.at[ibd], rq, s0)
            c1 = pltpu.async_copy(tk.at[ibs], rk, s1)
            c2 = pltpu.async_copy(tv.at[ibs], rv, s2)
            c0.wait()
            c1.wait()
            c2.wait()
            pltpu.sync_copy(rq, oq.at[pl.ds(off, CH)])
            pltpu.sync_copy(rk, ok.at[pl.ds(off, CH)])
            pltpu.sync_copy(rv, ov.at[pl.ds(off, CH)])
            return carry

        lax.fori_loop(0, NJ, step, 0)

    return gk(qt, kt, vt, dst, src)


def _sc_scatter(rows, dst, zeros_hbm):
    @functools.partial(
        pl.kernel,
        mesh=_mesh(),
        compiler_params=_SC_PARAMS,
        out_type=jax.ShapeDtypeStruct((NC * N, RW), jnp.float32),
        scratch_types=[
            pltpu.VMEM_SHARED((N, RW), jnp.float32),
            pltpu.VMEM((CH,), jnp.int32),
            pltpu.VMEM((CH, RW), jnp.float32),
        ],
    )
    def sk(rh, dh, zh, out, acc, ibuf, rbuf):
        cc = lax.axis_index("c")
        ss = lax.axis_index("s")
        base = (ss * NC + cc) * EPW
        row0 = ss * STRIPE
        pltpu.sync_copy(zh.at[pl.ds(row0, STRIPE)], acc.at[pl.ds(row0, STRIPE)])
        plsc.subcore_barrier()

        def step(j, carry):
            off = base + j * CH
            pltpu.sync_copy(dh.at[pl.ds(off, CH)], ibuf)
            pltpu.sync_copy(rh.at[pl.ds(off, CH)], rbuf)
            pltpu.sync_copy(rbuf, acc.at[ibuf], add=True)
            return carry

        lax.fori_loop(0, NJ, step, 0)
        plsc.subcore_barrier()
        pltpu.sync_copy(
            acc.at[pl.ds(row0, STRIPE)],
            out.at[pl.ds(cc * N + row0, STRIPE)],
        )

    return sk(rows, dst, zeros_hbm)


def _embed(pos8, wemb):
    def body(p_ref, w_ref, o_ref):
        o_ref[...] = jnp.tanh(
            jnp.dot(p_ref[...], w_ref[...], preferred_element_type=jnp.float32)
        )

    return pl.pallas_call(
        body,
        grid=(GN,),
        in_specs=[
            pl.BlockSpec((BN, 8), lambda i: (i, 0)),
            pl.BlockSpec((8, DP), lambda i: (0, 0)),
        ],
        out_specs=pl.BlockSpec((BN, DP), lambda i: (i, 0)),
        out_shape=jax.ShapeDtypeStruct((N, DP), jnp.float32),
    )(pos8, wemb)


def _geom(psrc, pdst):
    wid = np.float32(MAXR / NB)

    def body(a_ref, b_ref, rbf_ref, sh_ref):
        ci = lax.broadcasted_iota(jnp.int32, (1, 16), 1)
        centers = ci.astype(jnp.float32) * np.float32(MAXR / (NB - 1))
        cmask = (ci < NB).astype(jnp.float32)
        rel = b_ref[...] - a_ref[...]
        d2 = jnp.sum(rel * rel, axis=1, keepdims=True)
        dist = jnp.sqrt(d2) + 1e-9
        g = jnp.exp(-(((dist - centers) / wid) ** 2))
        env = jnp.exp(-d2 / (2.0 * MAXR * MAXR))
        rbf_ref[...] = g * env * cmask
        u = rel / dist
        x = u[:, 0:1]
        y = u[:, 1:2]
        z = u[:, 2:3]
        x2 = x * x
        y2 = y * y
        z2 = z * z
        sh_ref[...] = jnp.concatenate(
            [
                jnp.ones_like(x), x, y, z,
                x * y, y * z, 0.5 * (3.0 * z2 - 1.0), z * x,
                0.5 * (x2 - y2), y * (3.0 * x2 - y2), x * y * z,
                y * (5.0 * z2 - 1.0), z * (5.0 * z2 - 3.0),
                x * (5.0 * z2 - 1.0), z * (x2 - y2), x * (x2 - 3.0 * y2),
            ],
            axis=1,
        )

    return pl.pallas_call(
        body,
        grid=(GE,),
        in_specs=[
            pl.BlockSpec((BE, 16), lambda i: (i, 0)),
            pl.BlockSpec((BE, 16), lambda i: (i, 0)),
        ],
        out_specs=[
            pl.BlockSpec((BE, 16), lambda i: (i, 0)),
            pl.BlockSpec((BE, 16), lambda i: (i, 0)),
        ],
        out_shape=[
            jax.ShapeDtypeStruct((E, 16), jnp.float32),
            jax.ShapeDtypeStruct((E, 16), jnp.float32),
        ],
    )(psrc, pdst)


def _qkv(x, wq, wk, wv):
    def body(x_ref, wq_ref, wk_ref, wv_ref, q_ref, k_ref, v_ref):
        xv = x_ref[...]
        q_ref[...] = jnp.dot(xv, wq_ref[...], preferred_element_type=jnp.float32)
        k_ref[...] = jnp.dot(xv, wk_ref[...], preferred_element_type=jnp.float32)
        v_ref[...] = jnp.dot(xv, wv_ref[...], preferred_element_type=jnp.float32)

    wspec = pl.BlockSpec((DP, DP), lambda i: (0, 0))
    nspec = pl.BlockSpec((BN, DP), lambda i: (i, 0))
    nshape = jax.ShapeDtypeStruct((N, DP), jnp.float32)
    return pl.pallas_call(
        body,
        grid=(GN,),
        in_specs=[nspec, wspec, wspec, wspec],
        out_specs=[nspec, nspec, nspec],
        out_shape=[nshape, nshape, nshape],
    )(x, wq, wk, wv)


def _pass_a(qdst, kvsrc, rbfp, wr):
    def body(q_ref, k_ref, r_ref, w_ref, lg_ref, gm_ref, mx_ref):
        i = pl.program_id(0)
        rmod = jnp.dot(r_ref[...], w_ref[...], preferred_element_type=jnp.float32)
        prod = q_ref[...] * k_ref[...] * rmod
        lg = jnp.sum(prod, axis=1, keepdims=True) / _SQRT_D
        lg_ref[...] = lg
        bm = jnp.max(lg)

        @pl.when(i == 0)
        def _():
            mx_ref[0, 0] = bm

        @pl.when(i > 0)
        def _():
            mx_ref[0, 0] = jnp.maximum(mx_ref[0, 0], bm)

        gm_ref[0, 0] = mx_ref[0, 0]

    return pl.pallas_call(
        body,
        grid=(GE,),
        in_specs=[
            pl.BlockSpec((BE, DP), lambda i: (i, 0)),
            pl.BlockSpec((BE, DP), lambda i: (i, 0)),
            pl.BlockSpec((BE, 16), lambda i: (i, 0)),
            pl.BlockSpec((16, DP), lambda i: (0, 0)),
        ],
        out_specs=[
            pl.BlockSpec((BE, 1), lambda i: (i, 0)),
            pl.BlockSpec(memory_space=pltpu.SMEM),
        ],
        out_shape=[
            jax.ShapeDtypeStruct((E, 1), jnp.float32),
            jax.ShapeDtypeStruct((1, 1), jnp.float32),
        ],
        scratch_shapes=[pltpu.SMEM((1, 1), jnp.float32)],
    )(qdst, kvsrc, rbfp, wr)


def _pass_b(vsrc, rbfp, shp, logits, gmax, wr, wsh):
    def body(v_ref, r_ref, s_ref, lg_ref, gm_ref, wr_ref, ws_ref, o_ref):
        rmod = jnp.dot(r_ref[...], wr_ref[...], preferred_element_type=jnp.float32)
        smod = jnp.dot(s_ref[...], ws_ref[...], preferred_element_type=jnp.float32)
        ve = v_ref[...] * rmod + smod
        ex = jnp.exp(lg_ref[...] - gm_ref[0, 0])
        o_ref[...] = jnp.concatenate(
            [ve * ex, ex, jnp.zeros((BE, RW - DP - 1), jnp.float32)], axis=1
        )

    return pl.pallas_call(
        body,
        grid=(GE,),
        in_specs=[
            pl.BlockSpec((BE, DP), lambda i: (i, 0)),
            pl.BlockSpec((BE, 16), lambda i: (i, 0)),
            pl.BlockSpec((BE, 16), lambda i: (i, 0)),
            pl.BlockSpec((BE, 1), lambda i: (i, 0)),
            pl.BlockSpec(memory_space=pltpu.SMEM),
            pl.BlockSpec((16, DP), lambda i: (0, 0)),
            pl.BlockSpec((16, DP), lambda i: (0, 0)),
        ],
        out_specs=pl.BlockSpec((BE, RW), lambda i: (i, 0)),
        out_shape=jax.ShapeDtypeStruct((E, RW), jnp.float32),
    )(vsrc, rbfp, shp, logits, gmax, wr, wsh)


def _combine(agg2, x, wo):
    def body(a_ref, x_ref, w_ref, o_ref):
        a = a_ref[0] + a_ref[1]
        den = a[:, DP:DP + 1] + 1e-9
        node = a[:, :DP] / den
        xn = x_ref[...] + jnp.dot(node, w_ref[...], preferred_element_type=jnp.float32)
        mu = jnp.sum(xn, axis=1, keepdims=True) / D
        msk = (lax.broadcasted_iota(jnp.int32, (BN, DP), 1) < D).astype(jnp.float32)
        dv = (xn - mu) * msk
        var = jnp.sum(dv * dv, axis=1, keepdims=True) / D
        sig = jnp.sqrt(var) + 1e-5
        o_ref[...] = dv / sig

    return pl.pallas_call(
        body,
        grid=(GN,),
        in_specs=[
            pl.BlockSpec((2, BN, RW), lambda i: (0, i, 0)),
            pl.BlockSpec((BN, DP), lambda i: (i, 0)),
            pl.BlockSpec((DP, DP), lambda i: (0, 0)),
        ],
        out_specs=pl.BlockSpec((BN, DP), lambda i: (i, 0)),
        out_shape=jax.ShapeDtypeStruct((N, DP), jnp.float32),
    )(agg2, x, wo)


def _final(x, wout):
    def body(x_ref, w_ref, o_ref):
        o_ref[...] = jnp.dot(x_ref[...], w_ref[...], preferred_element_type=jnp.float32)

    return pl.pallas_call(
        body,
        grid=(GN,),
        in_specs=[
            pl.BlockSpec((BN, DP), lambda i: (i, 0)),
            pl.BlockSpec((DP, D), lambda i: (0, 0)),
        ],
        out_specs=pl.BlockSpec((BN, D), lambda i: (i, 0)),
        out_shape=jax.ShapeDtypeStruct((N, D), jnp.float32),
    )(x, wout)


def kernel(pos, edge_index, Wemb, Wq, Wk, Wv, Wr, Wsh, Wo, Wout):
    f32 = jnp.float32
    src = edge_index[0].astype(jnp.int32)
    dst = edge_index[1].astype(jnp.int32)

    pos8 = jnp.pad(pos, ((0, 0), (0, 5)))
    pos16 = jnp.pad(pos, ((0, 0), (0, 13)))
    wemb = jnp.pad(Wemb, ((0, 5), (0, DP - D)))
    wq = jnp.pad(Wq, ((0, 0), (0, DP - D), (0, DP - D)))
    wk = jnp.pad(Wk, ((0, 0), (0, DP - D), (0, DP - D)))
    wv = jnp.pad(Wv, ((0, 0), (0, DP - D), (0, DP - D)))
    wr = jnp.pad(Wr, ((0, 0), (0, 16 - NB), (0, DP - D)))
    wsh = jnp.pad(Wsh, ((0, 0), (0, 0), (0, DP - D)))
    wo = jnp.pad(Wo, ((0, 0), (0, DP - D), (0, DP - D)))
    wout = jnp.pad(Wout, ((0, DP - D), (0, 0)))
    zeros_rw = jnp.zeros((N, RW), f32)

    x = _embed(pos8, wemb)
    psrc, pdst = _sc_gather2(pos16, src, pos16, dst)
    rbfp, shp = _geom(psrc, pdst)

    for i in range(L):
        q, k, v = _qkv(x, wq[i], wk[i], wv[i])
        qdst, ksrc, vsrc = _sc_gather_qkv(q, k, v, dst, src)
        logits, gmax = _pass_a(qdst, ksrc, rbfp, wr[i])
        rows = _pass_b(vsrc, rbfp, shp, logits, gmax, wr[i], wsh[i])
        agg2 = _sc_scatter(rows, dst, zeros_rw)
        x = _combine(agg2.reshape(2, N, RW), x, wo[i])

    return _final(x, wout)


# speedup vs baseline: 5.6427x; 1.7188x over previous
import functools

import jax
import jax.numpy as jnp
import numpy as np
from jax import lax
from jax.experimental import pallas as pl
from jax.experimental.pallas import tpu as pltpu
from jax.experimental.pallas import tpu_sc as plsc

N = 10000
E = 320000
D = 86
DP = 128
EXC = 96
NB = 10
MAXR = 2.5
L = 3

NC = 2
NS = 16
NW = NC * NS
EPW = E // NW
CH = 80
NJ = EPW // CH
STRIPE = 1000

BE = 3200
GE = E // BE
BN = 2000
GN = N // BN

_SQRT_D = np.sqrt(D).astype(np.float32)


def _mesh():
    return plsc.VectorSubcoreMesh(core_axis_name="c", subcore_axis_name="s")


def _sc_gather_pos(px, py, pz, src, dst):
    evec = jax.ShapeDtypeStruct((E,), jnp.float32)

    @functools.partial(
        pl.kernel,
        mesh=_mesh(),
        out_type=[evec] * 6,
        scratch_types=[
            pltpu.VMEM((CH,), jnp.int32),
            pltpu.VMEM((CH,), jnp.int32),
            pltpu.VMEM((CH,), jnp.float32),
            pltpu.VMEM((CH,), jnp.float32),
            pltpu.VMEM((CH,), jnp.float32),
            pltpu.VMEM((CH,), jnp.float32),
            pltpu.VMEM((CH,), jnp.float32),
            pltpu.VMEM((CH,), jnp.float32),
            pltpu.SemaphoreType.DMA,
            pltpu.SemaphoreType.DMA,
        ],
    )
    def gk(tx, ty, tz, sr_, ds_, oxs, oys, ozs, oxd, oyd, ozd,
           ibs, ibd, bxs, bys, bzs, bxd, byd, bzd, s0, s1):
        cc = lax.axis_index("c")
        ss = lax.axis_index("s")
        base = (ss * NC + cc) * EPW

        def step(j, carry):
            off = base + j * CH
            pltpu.sync_copy(sr_.at[pl.ds(off, CH)], ibs)
            pltpu.sync_copy(ds_.at[pl.ds(off, CH)], ibd)
            c0 = pltpu.async_copy(tx.at[ibs], bxs, s0)
            c1 = pltpu.async_copy(ty.at[ibs], bys, s0)
            c2 = pltpu.async_copy(tz.at[ibs], bzs, s0)
            c3 = pltpu.async_copy(tx.at[ibd], bxd, s1)
            c4 = pltpu.async_copy(ty.at[ibd], byd, s1)
            c5 = pltpu.async_copy(tz.at[ibd], bzd, s1)
            c0.wait(); c1.wait(); c2.wait()
            c3.wait(); c4.wait(); c5.wait()
            pltpu.sync_copy(bxs, oxs.at[pl.ds(off, CH)])
            pltpu.sync_copy(bys, oys.at[pl.ds(off, CH)])
            pltpu.sync_copy(bzs, ozs.at[pl.ds(off, CH)])
            pltpu.sync_copy(bxd, oxd.at[pl.ds(off, CH)])
            pltpu.sync_copy(byd, oyd.at[pl.ds(off, CH)])
            pltpu.sync_copy(bzd, ozd.at[pl.ds(off, CH)])
            return carry

        lax.fori_loop(0, NJ, step, 0)

    return gk(px, py, pz, src, dst)


def _sc_gather_qkv(qt, kt, vt, dst, src):
    erows = jax.ShapeDtypeStruct((E, DP), jnp.float32)

    @functools.partial(
        pl.kernel,
        mesh=_mesh(),
        out_type=[erows, erows, erows],
        scratch_types=[
            pltpu.VMEM((CH,), jnp.int32),
            pltpu.VMEM((CH,), jnp.int32),
            pltpu.VMEM((CH, DP), jnp.float32),
            pltpu.VMEM((CH, DP), jnp.float32),
            pltpu.VMEM((CH, DP), jnp.float32),
            pltpu.SemaphoreType.DMA,
            pltpu.SemaphoreType.DMA,
            pltpu.SemaphoreType.DMA,
        ],
    )
    def gk(tq, tk, tv, ds_, sr_, oq, ok, ov, ibd, ibs, rq, rk, rv, s0, s1, s2):
        cc = lax.axis_index("c")
        ss = lax.axis_index("s")
        base = (ss * NC + cc) * EPW

        def step(j, carry):
            off = base + j * CH
            pltpu.sync_copy(ds_.at[pl.ds(off, CH)], ibd)
            pltpu.sync_copy(sr_.at[pl.ds(off, CH)], ibs)
            c0 = pltpu.async_copy(tq.at[ibd], rq, s0)
            c1 = pltpu.async_copy(tk.at[ibs], rk, s1)
            c2 = pltpu.async_copy(tv.at[ibs], rv, s2)
            c0.wait()
            c1.wait()
            c2.wait()
            pltpu.sync_copy(rq, oq.at[pl.ds(off, CH)])
            pltpu.sync_copy(rk, ok.at[pl.ds(off, CH)])
            pltpu.sync_copy(rv, ov.at[pl.ds(off, CH)])
            return carry

        lax.fori_loop(0, NJ, step, 0)

    return gk(qt, kt, vt, dst, src)


def _sc_scatter(rows, dst, zeros_hbm):
    @functools.partial(
        pl.kernel,
        mesh=_mesh(),
        out_type=jax.ShapeDtypeStruct((NC * N, DP), jnp.float32),
        scratch_types=[
            pltpu.VMEM_SHARED((N, DP), jnp.float32),
            pltpu.VMEM((CH,), jnp.int32),
            pltpu.VMEM((CH, DP), jnp.float32),
        ],
    )
    def sk(rh, dh, zh, out, acc, ibuf, rbuf):
        cc = lax.axis_index("c")
        ss = lax.axis_index("s")
        base = (ss * NC + cc) * EPW
        row0 = ss * STRIPE

        @pl.when(ss < N // STRIPE)
        def _():
            pltpu.sync_copy(zh.at[pl.ds(row0, STRIPE)], acc.at[pl.ds(row0, STRIPE)])

        plsc.subcore_barrier()

        def step(j, carry):
            off = base + j * CH
            pltpu.sync_copy(dh.at[pl.ds(off, CH)], ibuf)
            pltpu.sync_copy(rh.at[pl.ds(off, CH)], rbuf)
            pltpu.sync_copy(rbuf, acc.at[ibuf], add=True)
            return carry

        lax.fori_loop(0, NJ, step, 0)
        plsc.subcore_barrier()

        @pl.when(ss < N // STRIPE)
        def _():
            pltpu.sync_copy(
                acc.at[pl.ds(row0, STRIPE)],
                out.at[pl.ds(cc * N + row0, STRIPE)],
            )

    return sk(rows, dst, zeros_hbm)


def _embed(pos8, wemb):
    def body(p_ref, w_ref, o_ref):
        o_ref[...] = jnp.tanh(
            jnp.dot(p_ref[...], w_ref[...], preferred_element_type=jnp.float32)
        )

    return pl.pallas_call(
        body,
        grid=(GN,),
        in_specs=[
            pl.BlockSpec((BN, 8), lambda i: (i, 0)),
            pl.BlockSpec((8, DP), lambda i: (0, 0)),
        ],
        out_specs=pl.BlockSpec((BN, DP), lambda i: (i, 0)),
        out_shape=jax.ShapeDtypeStruct((N, DP), jnp.float32),
    )(pos8, wemb)


def _geom(xs, ys, zs, xd, yd, zd):
    wid = np.float32(MAXR / NB)

    def body(xs_r, ys_r, zs_r, xd_r, yd_r, zd_r, rbf_ref, sh_ref):
        rx = xd_r[0] - xs_r[0]
        ry = yd_r[0] - ys_r[0]
        rz = zd_r[0] - zs_r[0]
        d2 = rx * rx + ry * ry + rz * rz
        dist = jnp.sqrt(d2) + 1e-9
        env = jnp.exp(-d2 / (2.0 * MAXR * MAXR))
        rows = []
        for j in range(16):
            if j < NB:
                cj = np.float32(j * MAXR / (NB - 1))
                rows.append(jnp.exp(-(((dist - cj) / wid) ** 2)) * env)
            else:
                rows.append(jnp.zeros_like(dist))
        rbf_ref[...] = jnp.concatenate(rows, axis=0)
        inv = 1.0 / dist
        x = rx * inv
        y = ry * inv
        z = rz * inv
        x2 = x * x
        y2 = y * y
        z2 = z * z
        sh_ref[...] = jnp.concatenate(
            [
                jnp.ones_like(x), x, y, z,
                x * y, y * z, 0.5 * (3.0 * z2 - 1.0), z * x,
                0.5 * (x2 - y2), y * (3.0 * x2 - y2), x * y * z,
                y * (5.0 * z2 - 1.0), z * (5.0 * z2 - 3.0),
                x * (5.0 * z2 - 1.0), z * (x2 - y2), x * (x2 - 3.0 * y2),
            ],
            axis=0,
        )

    espec = pl.BlockSpec((1, 1, BE), lambda i: (i, 0, 0))
    tspec = pl.BlockSpec((16, BE), lambda i: (0, i))
    tshape = jax.ShapeDtypeStruct((16, E), jnp.float32)
    return pl.pallas_call(
        body,
        grid=(GE,),
        in_specs=[espec] * 6,
        out_specs=[tspec, tspec],
        out_shape=[tshape, tshape],
    )(xs, ys, zs, xd, yd, zd)


def _qkv(x, wq, wk, wv):
    def body(x_ref, wq_ref, wk_ref, wv_ref, q_ref, k_ref, v_ref):
        xv = x_ref[...]
        q_ref[...] = jnp.dot(xv, wq_ref[...], preferred_element_type=jnp.float32)
        k_ref[...] = jnp.dot(xv, wk_ref[...], preferred_element_type=jnp.float32)
        v_ref[...] = jnp.dot(xv, wv_ref[...], preferred_element_type=jnp.float32)

    wspec = pl.BlockSpec((DP, DP), lambda i: (0, 0))
    nspec = pl.BlockSpec((BN, DP), lambda i: (i, 0))
    nshape = jax.ShapeDtypeStruct((N, DP), jnp.float32)
    return pl.pallas_call(
        body,
        grid=(GN,),
        in_specs=[nspec, wspec, wspec, wspec],
        out_specs=[nspec, nspec, nspec],
        out_shape=[nshape, nshape, nshape],
    )(x, wq, wk, wv)


def _tmod(t_ref, w_ref):
    return lax.dot_general(
        t_ref[...], w_ref[...], (((0,), (0,)), ((), ())),
        preferred_element_type=jnp.float32,
    )


def _pass_a(qdst, ksrc, rbf_t, wr):
    def body(q_ref, k_ref, r_ref, w_ref, lg_ref, gm_ref, mx_ref):
        i = pl.program_id(0)
        rmod = _tmod(r_ref, w_ref)
        prod = q_ref[...] * k_ref[...] * rmod
        lg = jnp.sum(prod, axis=1, keepdims=True) / _SQRT_D
        lg_ref[...] = lg
        bm = jnp.max(lg)

        @pl.when(i == 0)
        def _():
            mx_ref[0, 0] = bm

        @pl.when(i > 0)
        def _():
            mx_ref[0, 0] = jnp.maximum(mx_ref[0, 0], bm)

        gm_ref[0, 0] = mx_ref[0, 0]

    return pl.pallas_call(
        body,
        grid=(GE,),
        in_specs=[
            pl.BlockSpec((BE, DP), lambda i: (i, 0)),
            pl.BlockSpec((BE, DP), lambda i: (i, 0)),
            pl.BlockSpec((16, BE), lambda i: (0, i)),
            pl.BlockSpec((16, DP), lambda i: (0, 0)),
        ],
        out_specs=[
            pl.BlockSpec((BE, 1), lambda i: (i, 0)),
            pl.BlockSpec(memory_space=pltpu.SMEM),
        ],
        out_shape=[
            jax.ShapeDtypeStruct((E, 1), jnp.float32),
            jax.ShapeDtypeStruct((1, 1), jnp.float32),
        ],
        scratch_shapes=[pltpu.SMEM((1, 1), jnp.float32)],
    )(qdst, ksrc, rbf_t, wr)


def _pass_b(vsrc, rbf_t, sh_t, logits, gmax, wr, wsh):
    def body(v_ref, r_ref, s_ref, lg_ref, gm_ref, wr_ref, ws_ref, o_ref):
        rmod = _tmod(r_ref, wr_ref)
        smod = _tmod(s_ref, ws_ref)
        ve = v_ref[...] * rmod + smod
        ex = jnp.exp(lg_ref[...] - gm_ref[0, 0])
        oh = (lax.broadcasted_iota(jnp.int32, (BE, DP), 1) == EXC).astype(
            jnp.float32
        )
        o_ref[...] = ex * (ve + oh)

    return pl.pallas_call(
        body,
        grid=(GE,),
        in_specs=[
            pl.BlockSpec((BE, DP), lambda i: (i, 0)),
            pl.BlockSpec((16, BE), lambda i: (0, i)),
            pl.BlockSpec((16, BE), lambda i: (0, i)),
            pl.BlockSpec((BE, 1), lambda i: (i, 0)),
            pl.BlockSpec(memory_space=pltpu.SMEM),
            pl.BlockSpec((16, DP), lambda i: (0, 0)),
            pl.BlockSpec((16, DP), lambda i: (0, 0)),
        ],
        out_specs=pl.BlockSpec((BE, DP), lambda i: (i, 0)),
        out_shape=jax.ShapeDtypeStruct((E, DP), jnp.float32),
    )(vsrc, rbf_t, sh_t, logits, gmax, wr, wsh)


def _combine(agg2, x, wo):
    def body(a_ref, x_ref, w_ref, o_ref):
        a = a_ref[0] + a_ref[1]
        den = a[:, EXC:EXC + 1] + 1e-9
        node = a / den
        xn = x_ref[...] + jnp.dot(node, w_ref[...], preferred_element_type=jnp.float32)
        mu = jnp.sum(xn, axis=1, keepdims=True) / D
        msk = (lax.broadcasted_iota(jnp.int32, (BN, DP), 1) < D).astype(jnp.float32)
        dv = (xn - mu) * msk
        var = jnp.sum(dv * dv, axis=1, keepdims=True) / D
        sig = jnp.sqrt(var) + 1e-5
        o_ref[...] = dv / sig

    return pl.pallas_call(
        body,
        grid=(GN,),
        in_specs=[
            pl.BlockSpec((2, BN, DP), lambda i: (0, i, 0)),
            pl.BlockSpec((BN, DP), lambda i: (i, 0)),
            pl.BlockSpec((DP, DP), lambda i: (0, 0)),
        ],
        out_specs=pl.BlockSpec((BN, DP), lambda i: (i, 0)),
        out_shape=jax.ShapeDtypeStruct((N, DP), jnp.float32),
    )(agg2, x, wo)


def _final(x, wout):
    def body(x_ref, w_ref, o_ref):
        o_ref[...] = jnp.dot(x_ref[...], w_ref[...], preferred_element_type=jnp.float32)

    return pl.pallas_call(
        body,
        grid=(GN,),
        in_specs=[
            pl.BlockSpec((BN, DP), lambda i: (i, 0)),
            pl.BlockSpec((DP, D), lambda i: (0, 0)),
        ],
        out_specs=pl.BlockSpec((BN, D), lambda i: (i, 0)),
        out_shape=jax.ShapeDtypeStruct((N, D), jnp.float32),
    )(x, wout)


def kernel(pos, edge_index, Wemb, Wq, Wk, Wv, Wr, Wsh, Wo, Wout):
    f32 = jnp.float32
    src = edge_index[0].astype(jnp.int32)
    dst = edge_index[1].astype(jnp.int32)

    pos8 = jnp.pad(pos, ((0, 0), (0, 5)))
    px = jnp.asarray(pos[:, 0], f32)
    py = jnp.asarray(pos[:, 1], f32)
    pz = jnp.asarray(pos[:, 2], f32)
    wemb = jnp.pad(Wemb, ((0, 5), (0, DP - D)))
    wq = jnp.pad(Wq, ((0, 0), (0, DP - D), (0, DP - D)))
    wk = jnp.pad(Wk, ((0, 0), (0, DP - D), (0, DP - D)))
    wv = jnp.pad(Wv, ((0, 0), (0, DP - D), (0, DP - D)))
    wr = jnp.pad(Wr, ((0, 0), (0, 16 - NB), (0, DP - D)))
    wsh = jnp.pad(Wsh, ((0, 0), (0, 0), (0, DP - D)))
    wo = jnp.pad(Wo, ((0, 0), (0, DP - D), (0, DP - D)))
    wout = jnp.pad(Wout, ((0, DP - D), (0, 0)))
    zeros_dp = jnp.zeros((N, DP), f32)

    x = _embed(pos8, wemb)
    xs, ys, zs, xd, yd, zd = _sc_gather_pos(px, py, pz, src, dst)
    shp3 = (GE, 1, BE)
    rbf_t, sh_t = _geom(
        xs.reshape(shp3), ys.reshape(shp3), zs.reshape(shp3),
        xd.reshape(shp3), yd.reshape(shp3), zd.reshape(shp3),
    )

    for i in range(L):
        q, k, v = _qkv(x, wq[i], wk[i], wv[i])
        qdst, ksrc, vsrc = _sc_gather_qkv(q, k, v, dst, src)
        logits, gmax = _pass_a(qdst, ksrc, rbf_t, wr[i])
        rows = _pass_b(vsrc, rbf_t, sh_t, logits, gmax, wr[i], wsh[i])
        agg2 = _sc_scatter(rows, dst, zeros_dp)
        x = _combine(agg2.reshape(2, N, DP), x, wo[i])

    return _final(x, wout)


# speedup vs baseline: 7.7405x; 1.3718x over previous
import functools

import jax
import jax.numpy as jnp
import numpy as np
from jax import lax
from jax.experimental import pallas as pl
from jax.experimental.pallas import tpu as pltpu
from jax.experimental.pallas import tpu_sc as plsc

N = 10000
E = 320000
D = 86
DP = 128
EXC = 96
NB = 10
MAXR = 2.5
L = 3

NC = 2
NS = 16
NW = NC * NS
EPW = E // NW
CH = 80
NJ = EPW // CH
STRIPE = 1000

BE = 3200
GE = E // BE
BN = 2000
GN = N // BN

_SQRT_D = np.sqrt(D).astype(np.float32)


def _mesh():
    return plsc.VectorSubcoreMesh(core_axis_name="c", subcore_axis_name="s")


def _sc_gather_pos(px, py, pz, src, dst):
    evec = jax.ShapeDtypeStruct((E,), jnp.float32)
    fbuf = pltpu.VMEM((CH,), jnp.float32)
    ibuf = pltpu.VMEM((CH,), jnp.int32)

    @functools.partial(
        pl.kernel,
        mesh=_mesh(),
        out_type=[evec] * 6,
        scratch_types=[ibuf] * 4 + [fbuf] * 12
        + [pltpu.SemaphoreType.DMA] * 5,
    )
    def gk(tx, ty, tz, sr_, ds_, *refs):
        outs = refs[0:6]
        ib = (refs[6:8], refs[8:10])
        rows = (refs[10:16], refs[16:22])
        si = (refs[22], refs[23])
        sg = refs[24]
        sw = (refs[25], refs[26])
        tabs = (tx, ty, tz, tx, ty, tz)
        cc = lax.axis_index("c")
        ss = lax.axis_index("s")
        base = (ss * NC + cc) * EPW

        def idx_fetch(j, p):
            off = base + j * CH
            pltpu.async_copy(sr_.at[pl.ds(off, CH)], ib[p][0], si[p])
            pltpu.async_copy(ds_.at[pl.ds(off, CH)], ib[p][1], si[p])

        def chunk(j, p, drain):
            off = base + j * CH
            pltpu.make_async_copy(sr_.at[pl.ds(0, CH)], ib[p][0], si[p]).wait()
            pltpu.make_async_copy(ds_.at[pl.ds(0, CH)], ib[p][1], si[p]).wait()
            if drain:
                for q in range(6):
                    pltpu.make_async_copy(
                        rows[p][q], outs[q].at[pl.ds(0, CH)], sw[p]).wait()
            cs = [pltpu.async_copy(tabs[q].at[ib[p][q // 3]], rows[p][q], sg)
                  for q in range(6)]
            for c in cs:
                c.wait()
            for q in range(6):
                pltpu.async_copy(rows[p][q], outs[q].at[pl.ds(off, CH)], sw[p])

        idx_fetch(0, 0)
        idx_fetch(1, 1)
        chunk(0, 0, False)
        idx_fetch(2, 0)
        chunk(1, 1, False)
        idx_fetch(3, 1)
        chunk(2, 0, True)

        def body(t, carry):
            j = 2 * t + 1
            idx_fetch(j + 1, 0)
            chunk(j, 1, True)
            idx_fetch(j + 2, 1)
            chunk(j + 1, 0, True)
            return carry

        lax.fori_loop(1, (NJ - 3) // 2, body, 0)
        idx_fetch(NJ - 1, 0)
        chunk(NJ - 2, 1, True)
        chunk(NJ - 1, 0, True)
        for p in (1, 0):
            for q in range(6):
                pltpu.make_async_copy(
                    rows[p][q], outs[q].at[pl.ds(0, CH)], sw[p]).wait()

    return gk(px, py, pz, src, dst)


def _sc_gather_qkv(qt, kt, vt, dst, src):
    erows = jax.ShapeDtypeStruct((E, DP), jnp.float32)
    rbuf = pltpu.VMEM((CH, DP), jnp.float32)
    ibuf = pltpu.VMEM((CH,), jnp.int32)

    @functools.partial(
        pl.kernel,
        mesh=_mesh(),
        out_type=[erows, erows, erows],
        scratch_types=[ibuf] * 4 + [rbuf] * 6 + [pltpu.SemaphoreType.DMA] * 5,
    )
    def gk(tq, tk, tv, ds_, sr_, *refs):
        outs = refs[0:3]
        ib = (refs[3:5], refs[5:7])
        rows = (refs[7:10], refs[10:13])
        si = (refs[13], refs[14])
        sg = refs[15]
        sw = (refs[16], refs[17])
        tabs = (tq, tk, tv)
        cc = lax.axis_index("c")
        ss = lax.axis_index("s")
        base = (ss * NC + cc) * EPW

        def idx_fetch(j, p):
            off = base + j * CH
            pltpu.async_copy(ds_.at[pl.ds(off, CH)], ib[p][0], si[p])
            pltpu.async_copy(sr_.at[pl.ds(off, CH)], ib[p][1], si[p])

        def chunk(j, p, drain):
            off = base + j * CH
            pltpu.make_async_copy(ds_.at[pl.ds(0, CH)], ib[p][0], si[p]).wait()
            pltpu.make_async_copy(sr_.at[pl.ds(0, CH)], ib[p][1], si[p]).wait()
            if drain:
                for q in range(3):
                    pltpu.make_async_copy(
                        rows[p][q], outs[q].at[pl.ds(0, CH)], sw[p]).wait()
            cs = [pltpu.async_copy(tabs[q].at[ib[p][min(q, 1)]], rows[p][q], sg)
                  for q in range(3)]
            for c in cs:
                c.wait()
            for q in range(3):
                pltpu.async_copy(rows[p][q], outs[q].at[pl.ds(off, CH)], sw[p])

        idx_fetch(0, 0)
        idx_fetch(1, 1)
        chunk(0, 0, False)
        idx_fetch(2, 0)
        chunk(1, 1, False)
        idx_fetch(3, 1)
        chunk(2, 0, True)

        def body(t, carry):
            j = 2 * t + 1
            idx_fetch(j + 1, 0)
            chunk(j, 1, True)
            idx_fetch(j + 2, 1)
            chunk(j + 1, 0, True)
            return carry

        lax.fori_loop(1, (NJ - 3) // 2, body, 0)
        idx_fetch(NJ - 1, 0)
        chunk(NJ - 2, 1, True)
        chunk(NJ - 1, 0, True)
        for p in (1, 0):
            for q in range(3):
                pltpu.make_async_copy(
                    rows[p][q], outs[q].at[pl.ds(0, CH)], sw[p]).wait()

    return gk(qt, kt, vt, dst, src)


def _sc_scatter(rows, dst, zeros_hbm):
    @functools.partial(
        pl.kernel,
        mesh=_mesh(),
        out_type=jax.ShapeDtypeStruct((NC * N, DP), jnp.float32),
        scratch_types=[
            pltpu.VMEM_SHARED((N, DP), jnp.float32),
            pltpu.VMEM((CH,), jnp.int32),
            pltpu.VMEM((CH,), jnp.int32),
            pltpu.VMEM((CH, DP), jnp.float32),
            pltpu.VMEM((CH, DP), jnp.float32),
            pltpu.SemaphoreType.DMA,
            pltpu.SemaphoreType.DMA,
        ],
    )
    def sk(rh, dh, zh, out, acc, ib0, ib1, rb0, rb1, sl0, sl1):
        ib = (ib0, ib1)
        rb = (rb0, rb1)
        sl = (sl0, sl1)
        cc = lax.axis_index("c")
        ss = lax.axis_index("s")
        base = (ss * NC + cc) * EPW
        row0 = ss * STRIPE

        @pl.when(ss < N // STRIPE)
        def _():
            pltpu.sync_copy(zh.at[pl.ds(row0, STRIPE)], acc.at[pl.ds(row0, STRIPE)])

        plsc.subcore_barrier()

        def fetch(j, p):
            off = base + j * CH
            pltpu.async_copy(dh.at[pl.ds(off, CH)], ib[p], sl[p])
            pltpu.async_copy(rh.at[pl.ds(off, CH)], rb[p], sl[p])

        def sadd(p):
            pltpu.make_async_copy(dh.at[pl.ds(0, CH)], ib[p], sl[p]).wait()
            pltpu.make_async_copy(rh.at[pl.ds(0, CH)], rb[p], sl[p]).wait()
            pltpu.sync_copy(rb[p], acc.at[ib[p]], add=True)

        fetch(0, 0)

        def step(t, carry):
            j = 2 * t
            fetch(j + 1, 1)
            sadd(0)
            fetch(j + 2, 0)
            sadd(1)
            return carry

        lax.fori_loop(0, (NJ - 1) // 2, step, 0)
        sadd(0)
        plsc.subcore_barrier()

        @pl.when(ss < N // STRIPE)
        def _():
            pltpu.sync_copy(
                acc.at[pl.ds(row0, STRIPE)],
                out.at[pl.ds(cc * N + row0, STRIPE)],
            )

    return sk(rows, dst, zeros_hbm)


def _embed(pos8, wemb):
    def body(p_ref, w_ref, o_ref):
        o_ref[...] = jnp.tanh(
            jnp.dot(p_ref[...], w_ref[...], preferred_element_type=jnp.float32)
        )

    return pl.pallas_call(
        body,
        grid=(GN,),
        in_specs=[
            pl.BlockSpec((BN, 8), lambda i: (i, 0)),
            pl.BlockSpec((8, DP), lambda i: (0, 0)),
        ],
        out_specs=pl.BlockSpec((BN, DP), lambda i: (i, 0)),
        out_shape=jax.ShapeDtypeStruct((N, DP), jnp.float32),
    )(pos8, wemb)


def _geom(xs, ys, zs, xd, yd, zd):
    wid = np.float32(MAXR / NB)

    def body(xs_r, ys_r, zs_r, xd_r, yd_r, zd_r, rbf_ref, sh_ref):
        rx = xd_r[0] - xs_r[0]
        ry = yd_r[0] - ys_r[0]
        rz = zd_r[0] - zs_r[0]
        d2 = rx * rx + ry * ry + rz * rz
        dist = jnp.sqrt(d2) + 1e-9
        env = jnp.exp(-d2 / (2.0 * MAXR * MAXR))
        rows = []
        for j in range(16):
            if j < NB:
                cj = np.float32(j * MAXR / (NB - 1))
                rows.append(jnp.exp(-(((dist - cj) / wid) ** 2)) * env)
            else:
                rows.append(jnp.zeros_like(dist))
        rbf_ref[...] = jnp.concatenate(rows, axis=0)
        inv = 1.0 / dist
        x = rx * inv
        y = ry * inv
        z = rz * inv
        x2 = x * x
        y2 = y * y
        z2 = z * z
        sh_ref[...] = jnp.concatenate(
            [
                jnp.ones_like(x), x, y, z,
                x * y, y * z, 0.5 * (3.0 * z2 - 1.0), z * x,
                0.5 * (x2 - y2), y * (3.0 * x2 - y2), x * y * z,
                y * (5.0 * z2 - 1.0), z * (5.0 * z2 - 3.0),
                x * (5.0 * z2 - 1.0), z * (x2 - y2), x * (x2 - 3.0 * y2),
            ],
            axis=0,
        )

    espec = pl.BlockSpec((1, 1, BE), lambda i: (i, 0, 0))
    tspec = pl.BlockSpec((16, BE), lambda i: (0, i))
    tshape = jax.ShapeDtypeStruct((16, E), jnp.float32)
    return pl.pallas_call(
        body,
        grid=(GE,),
        in_specs=[espec] * 6,
        out_specs=[tspec, tspec],
        out_shape=[tshape, tshape],
    )(xs, ys, zs, xd, yd, zd)


def _qkv(x, wq, wk, wv):
    def body(x_ref, wq_ref, wk_ref, wv_ref, q_ref, k_ref, v_ref):
        xv = x_ref[...]
        q_ref[...] = jnp.dot(xv, wq_ref[...], preferred_element_type=jnp.float32)
        k_ref[...] = jnp.dot(xv, wk_ref[...], preferred_element_type=jnp.float32)
        v_ref[...] = jnp.dot(xv, wv_ref[...], preferred_element_type=jnp.float32)

    wspec = pl.BlockSpec((DP, DP), lambda i: (0, 0))
    nspec = pl.BlockSpec((BN, DP), lambda i: (i, 0))
    nshape = jax.ShapeDtypeStruct((N, DP), jnp.float32)
    return pl.pallas_call(
        body,
        grid=(GN,),
        in_specs=[nspec, wspec, wspec, wspec],
        out_specs=[nspec, nspec, nspec],
        out_shape=[nshape, nshape, nshape],
    )(x, wq, wk, wv)


def _tmod(t_ref, w_ref):
    return lax.dot_general(
        t_ref[...], w_ref[...], (((0,), (0,)), ((), ())),
        preferred_element_type=jnp.float32,
    )


def _pass_a(qdst, ksrc, rbf_t, wr):
    def body(q_ref, k_ref, r_ref, w_ref, lg_ref, gm_ref, mx_ref):
        i = pl.program_id(0)
        rmod = _tmod(r_ref, w_ref)
        prod = q_ref[...] * k_ref[...] * rmod
        lg = jnp.sum(prod, axis=1, keepdims=True) / _SQRT_D
        lg_ref[...] = lg
        bm = jnp.max(lg)

        @pl.when(i == 0)
        def _():
            mx_ref[0, 0] = bm

        @pl.when(i > 0)
        def _():
            mx_ref[0, 0] = jnp.maximum(mx_ref[0, 0], bm)

        gm_ref[0, 0] = mx_ref[0, 0]

    return pl.pallas_call(
        body,
        grid=(GE,),
        in_specs=[
            pl.BlockSpec((BE, DP), lambda i: (i, 0)),
            pl.BlockSpec((BE, DP), lambda i: (i, 0)),
            pl.BlockSpec((16, BE), lambda i: (0, i)),
            pl.BlockSpec((16, DP), lambda i: (0, 0)),
        ],
        out_specs=[
            pl.BlockSpec((BE, 1), lambda i: (i, 0)),
            pl.BlockSpec(memory_space=pltpu.SMEM),
        ],
        out_shape=[
            jax.ShapeDtypeStruct((E, 1), jnp.float32),
            jax.ShapeDtypeStruct((1, 1), jnp.float32),
        ],
        scratch_shapes=[pltpu.SMEM((1, 1), jnp.float32)],
    )(qdst, ksrc, rbf_t, wr)


def _pass_b(vsrc, rbf_t, sh_t, logits, gmax, wr, wsh):
    def body(v_ref, r_ref, s_ref, lg_ref, gm_ref, wr_ref, ws_ref, o_ref):
        rmod = _tmod(r_ref, wr_ref)
        smod = _tmod(s_ref, ws_ref)
        ve = v_ref[...] * rmod + smod
        ex = jnp.exp(lg_ref[...] - gm_ref[0, 0])
        oh = (lax.broadcasted_iota(jnp.int32, (BE, DP), 1) == EXC).astype(
            jnp.float32
        )
        o_ref[...] = ex * (ve + oh)

    return pl.pallas_call(
        body,
        grid=(GE,),
        in_specs=[
            pl.BlockSpec((BE, DP), lambda i: (i, 0)),
            pl.BlockSpec((16, BE), lambda i: (0, i)),
            pl.BlockSpec((16, BE), lambda i: (0, i)),
            pl.BlockSpec((BE, 1), lambda i: (i, 0)),
            pl.BlockSpec(memory_space=pltpu.SMEM),
            pl.BlockSpec((16, DP), lambda i: (0, 0)),
            pl.BlockSpec((16, DP), lambda i: (0, 0)),
        ],
        out_specs=pl.BlockSpec((BE, DP), lambda i: (i, 0)),
        out_shape=jax.ShapeDtypeStruct((E, DP), jnp.float32),
    )(vsrc, rbf_t, sh_t, logits, gmax, wr, wsh)


def _combine(agg2, x, wo):
    def body(a_ref, x_ref, w_ref, o_ref):
        a = a_ref[0] + a_ref[1]
        den = a[:, EXC:EXC + 1] + 1e-9
        node = a / den
        xn = x_ref[...] + jnp.dot(node, w_ref[...], preferred_element_type=jnp.float32)
        mu = jnp.sum(xn, axis=1, keepdims=True) / D
        msk = (lax.broadcasted_iota(jnp.int32, (BN, DP), 1) < D).astype(jnp.float32)
        dv = (xn - mu) * msk
        var = jnp.sum(dv * dv, axis=1, keepdims=True) / D
        sig = jnp.sqrt(var) + 1e-5
        o_ref[...] = dv / sig

    return pl.pallas_call(
        body,
        grid=(GN,),
        in_specs=[
            pl.BlockSpec((2, BN, DP), lambda i: (0, i, 0)),
            pl.BlockSpec((BN, DP), lambda i: (i, 0)),
            pl.BlockSpec((DP, DP), lambda i: (0, 0)),
        ],
        out_specs=pl.BlockSpec((BN, DP), lambda i: (i, 0)),
        out_shape=jax.ShapeDtypeStruct((N, DP), jnp.float32),
    )(agg2, x, wo)


def _final(x, wout):
    def body(x_ref, w_ref, o_ref):
        o_ref[...] = jnp.dot(x_ref[...], w_ref[...], preferred_element_type=jnp.float32)

    return pl.pallas_call(
        body,
        grid=(GN,),
        in_specs=[
            pl.BlockSpec((BN, DP), lambda i: (i, 0)),
            pl.BlockSpec((DP, D), lambda i: (0, 0)),
        ],
        out_specs=pl.BlockSpec((BN, D), lambda i: (i, 0)),
        out_shape=jax.ShapeDtypeStruct((N, D), jnp.float32),
    )(x, wout)


def kernel(pos, edge_index, Wemb, Wq, Wk, Wv, Wr, Wsh, Wo, Wout):
    f32 = jnp.float32
    src = edge_index[0].astype(jnp.int32)
    dst = edge_index[1].astype(jnp.int32)

    pos8 = jnp.pad(pos, ((0, 0), (0, 5)))
    px = jnp.asarray(pos[:, 0], f32)
    py = jnp.asarray(pos[:, 1], f32)
    pz = jnp.asarray(pos[:, 2], f32)
    wemb = jnp.pad(Wemb, ((0, 5), (0, DP - D)))
    wq = jnp.pad(Wq, ((0, 0), (0, DP - D), (0, DP - D)))
    wk = jnp.pad(Wk, ((0, 0), (0, DP - D), (0, DP - D)))
    wv = jnp.pad(Wv, ((0, 0), (0, DP - D), (0, DP - D)))
    wr = jnp.pad(Wr, ((0, 0), (0, 16 - NB), (0, DP - D)))
    wsh = jnp.pad(Wsh, ((0, 0), (0, 0), (0, DP - D)))
    wo = jnp.pad(Wo, ((0, 0), (0, DP - D), (0, DP - D)))
    wout = jnp.pad(Wout, ((0, DP - D), (0, 0)))
    zeros_dp = jnp.zeros((N, DP), f32)

    x = _embed(pos8, wemb)
    xs, ys, zs, xd, yd, zd = _sc_gather_pos(px, py, pz, src, dst)
    shp3 = (GE, 1, BE)
    rbf_t, sh_t = _geom(
        xs.reshape(shp3), ys.reshape(shp3), zs.reshape(shp3),
        xd.reshape(shp3), yd.reshape(shp3), zd.reshape(shp3),
    )

    for i in range(L):
        q, k, v = _qkv(x, wq[i], wk[i], wv[i])
        qdst, ksrc, vsrc = _sc_gather_qkv(q, k, v, dst, src)
        logits, gmax = _pass_a(qdst, ksrc, rbf_t, wr[i])
        rows = _pass_b(vsrc, rbf_t, sh_t, logits, gmax, wr[i], wsh[i])
        agg2 = _sc_scatter(rows, dst, zeros_dp)
        x = _combine(agg2.reshape(2, N, DP), x, wo[i])

    return _final(x, wout)
